# baseline (speedup 1.0000x reference)
import functools

import jax
import jax.numpy as jnp
from jax import lax
from jax.experimental import pallas as pl
from jax.experimental.pallas import tpu as pltpu
from jax.experimental.pallas import tpu_sc as plsc

NC = 2
NS = 16
NW = NC * NS
L = 16

B, SEQ = 16384, 200
N_IDS = B * SEQ
JT, IT, SUB, LANE = 25, 128, 8, 128
KIT = 16
CHUNK = KIT * SUB * LANE
SEG = KIT * 2 * LANE
N_CHUNKS = N_IDS // CHUNK
MAX_SLOTS = 8
OUT_JT = SUB * IT * 2 * LANE


def _gather(arr, idx):
    return arr.at[idx].get(mode="promise_in_bounds")


@jax.jit
def _sc_lookup(ids_flat, emb_flat, w_flat, bias):
    mesh = plsc.VectorSubcoreMesh(core_axis_name="c", subcore_axis_name="s")

    @functools.partial(
        pl.kernel,
        mesh=mesh,
        out_type=jax.ShapeDtypeStruct((2 * N_IDS,), jnp.float32),
        scratch_types=[
            pltpu.VMEM((CHUNK,), jnp.int32),
            pltpu.VMEM((CHUNK,), jnp.int32),
            pltpu.VMEM((SUB * SEG,), jnp.float32),
            pltpu.VMEM((SUB * SEG,), jnp.float32),
            pltpu.VMEM((32,), jnp.float32),
            pltpu.VMEM((L,), jnp.float32),
            pltpu.VMEM((L,), jnp.float32),
            pltpu.SemaphoreType.DMA,
            pltpu.SemaphoreType.DMA,
            pltpu.SemaphoreType.DMA,
            pltpu.SemaphoreType.DMA,
        ],
    )
    def run(ids_hbm, emb_hbm, w_hbm, b_hbm, out_hbm, ids_v0, ids_v1,
            out_v0, out_v1, emb_v, w_v, b_v, si0, si1, so0, so1):
        wid = lax.axis_index("s") * NC + lax.axis_index("c")
        start = 6 * wid + jnp.minimum(wid, 8)
        cnt = jnp.where(wid < 8, 7, 6)
        ids_bufs, out_bufs = [ids_v0, ids_v1], [out_v0, out_v1]
        isems, osems = [si0, si1], [so0, so1]

        def in_copy(slot, issue):
            bsel = slot % 2
            cp = pltpu.make_async_copy(
                ids_hbm.at[pl.ds((start + slot) * CHUNK, CHUNK)],
                ids_bufs[bsel], isems[bsel])
            if issue:
                cp.start()
            return cp

        in_copy(0, True)
        in_copy(1, True)
        pltpu.sync_copy(emb_hbm, emb_v)
        pltpu.sync_copy(w_hbm, w_v.at[pl.ds(0, 8)])
        pltpu.sync_copy(b_hbm, b_v.at[pl.ds(0, 2)])

        iota = lax.iota(jnp.int32, L)
        e0 = emb_v[pl.ds(0, L)]
        e1 = emb_v[pl.ds(16, L)]
        wv = w_v[...]
        bv = b_v[...]
        v = iota & 7
        p = iota >> 3
        acc = _gather(bv, p)
        for d in range(4):
            e = _gather(e0 if d < 2 else e1, v + 8 * (d & 1))
            w = _gather(wv, p * 4 + d)
            acc = acc + e * w
        t0 = acc
        t1 = _gather(acc, (iota & 7) + 8)

        def out_copies(slot, issue):
            bsel = slot % 2
            q = start + slot
            jt = q >> 3
            it0 = (q & 7) * KIT
            cps = []
            for s in range(SUB):
                cp = pltpu.make_async_copy(
                    out_bufs[bsel].at[pl.ds(s * SEG, SEG)],
                    out_hbm.at[pl.ds(jt * OUT_JT + s * (IT * 2 * LANE)
                                     + it0 * 2 * LANE, SEG)],
                    osems[bsel])
                if issue:
                    cp.start()
                cps.append(cp)
            return cps

        for slot in range(MAX_SLOTS):
            bsel = slot % 2
            ids_v, out_v = ids_bufs[bsel], out_bufs[bsel]

            @pl.when(slot < cnt)
            def _(slot=slot, bsel=bsel, ids_v=ids_v, out_v=out_v):
                in_copy(slot, False).wait()
                if slot >= 2:
                    for cp in out_copies(slot - 2, False):
                        cp.wait()

                @pl.loop(0, KIT)
                def _(n):
                    for s in range(SUB):
                        off_in = n * 1024 + s * 128
                        off_out = s * SEG + n * 256
                        idvs = [ids_v[pl.ds(off_in + u * L, L)]
                                for u in range(8)]
                        for u in range(8):
                            out_v[pl.ds(off_out + u * L, L)] = (
                                _gather(t0, idvs[u]))
                        for u in range(8):
                            out_v[pl.ds(off_out + 128 + u * L, L)] = (
                                _gather(t1, idvs[u]))

                out_copies(slot, True)
                if slot + 2 < MAX_SLOTS:
                    @pl.when(slot + 2 < cnt)
                    def _(slot=slot):
                        in_copy(slot + 2, True)

        for slot in range(MAX_SLOTS):
            @pl.when((slot < cnt) & (slot + 2 >= cnt))
            def _(slot=slot):
                for cp in out_copies(slot, False):
                    cp.wait()

    return run(ids_flat, emb_flat, w_flat, bias)


def kernel(input_ids, embed_weight, lm_head_weight, lm_head_bias):
    ids_flat = (
        input_ids.astype(jnp.int32)
        .reshape(IT, LANE, JT, SUB)
        .transpose(2, 0, 3, 1)
        .reshape(-1)
    )
    emb_flat = embed_weight.T.reshape(-1)
    w_flat = lm_head_weight.reshape(-1)
    out_flat = _sc_lookup(ids_flat, emb_flat, w_flat, lm_head_bias)
    return (
        out_flat.reshape(SEQ, B // 128, 2, 128)
        .transpose(1, 3, 0, 2)
        .reshape(B, SEQ, 2)
    )


# speedup vs baseline: 1.0396x; 1.0396x over previous
import functools

import jax
import jax.numpy as jnp
from jax import lax
from jax.experimental import pallas as pl
from jax.experimental.pallas import tpu as pltpu
from jax.experimental.pallas import tpu_sc as plsc

NC = 2
NS = 16
NW = NC * NS
L = 16

B, SEQ = 16384, 200
N_IDS = B * SEQ
JT, IT, SUB, LANE = 25, 128, 8, 128
KIT = 16
CHUNK = KIT * SUB * LANE
SEG = KIT * 2 * LANE
N_CHUNKS = N_IDS // CHUNK
OUT_JT = SUB * IT * 2 * LANE


def _gather(arr, idx):
    return arr.at[idx].get(mode="promise_in_bounds")


@jax.jit
def _sc_lookup(ids_flat, emb_flat, w_flat, bias):
    mesh = plsc.VectorSubcoreMesh(core_axis_name="c", subcore_axis_name="s")

    @functools.partial(
        pl.kernel,
        mesh=mesh,
        out_type=jax.ShapeDtypeStruct((2 * N_IDS,), jnp.float32),
        scratch_types=[
            pltpu.VMEM((CHUNK,), jnp.int32),
            pltpu.VMEM((CHUNK,), jnp.int32),
            pltpu.VMEM((SUB * SEG,), jnp.float32),
            pltpu.VMEM((SUB * SEG,), jnp.float32),
            pltpu.VMEM((32,), jnp.float32),
            pltpu.VMEM((L,), jnp.float32),
            pltpu.VMEM((L,), jnp.float32),
            pltpu.SemaphoreType.DMA,
            pltpu.SemaphoreType.DMA,
            pltpu.SemaphoreType.DMA,
            pltpu.SemaphoreType.DMA,
        ],
    )
    def run(ids_hbm, emb_hbm, w_hbm, b_hbm, out_hbm, ids_v0, ids_v1,
            out_v0, out_v1, emb_v, w_v, b_v, si0, si1, so0, so1):
        wid = lax.axis_index("s") * NC + lax.axis_index("c")
        ids_bufs, out_bufs = [ids_v0, ids_v1], [out_v0, out_v1]
        isems, osems = [si0, si1], [so0, so1]

        def slot_params(slot):
            if slot < 6:
                q = wid * 6 + slot
                return q * CHUNK, KIT, q >> 3, (q & 7) * KIT
            qq = 192 + (wid >> 2)
            sub4 = (wid & 3) * 4
            return qq * CHUNK + sub4 * 1024, 4, 24, (qq & 7) * KIT + sub4

        def in_copy(slot, issue):
            bsel = slot % 2
            off, nb, _, _ = slot_params(slot)
            cp = pltpu.make_async_copy(
                ids_hbm.at[pl.ds(off, nb * 1024)],
                ids_bufs[bsel].at[pl.ds(0, nb * 1024)], isems[bsel])
            if issue:
                cp.start()
            return cp

        in_copy(0, True)
        in_copy(1, True)
        pltpu.sync_copy(emb_hbm, emb_v)
        pltpu.sync_copy(w_hbm, w_v.at[pl.ds(0, 8)])
        pltpu.sync_copy(b_hbm, b_v.at[pl.ds(0, 2)])

        iota = lax.iota(jnp.int32, L)
        e0 = emb_v[pl.ds(0, L)]
        e1 = emb_v[pl.ds(16, L)]
        wv = w_v[...]
        bv = b_v[...]
        v = iota & 7
        p = iota >> 3
        acc = _gather(bv, p)
        for d in range(4):
            e = _gather(e0 if d < 2 else e1, v + 8 * (d & 1))
            w = _gather(wv, p * 4 + d)
            acc = acc + e * w
        t0 = acc
        t1 = _gather(acc, (iota & 7) + 8)

        def out_copies(slot, issue):
            bsel = slot % 2
            _, nb, jt, it0 = slot_params(slot)
            seg = nb * 2 * LANE
            cps = []
            for s in range(SUB):
                cp = pltpu.make_async_copy(
                    out_bufs[bsel].at[pl.ds(s * seg, seg)],
                    out_hbm.at[pl.ds(jt * OUT_JT + s * (IT * 2 * LANE)
                                     + it0 * 2 * LANE, seg)],
                    osems[bsel])
                if issue:
                    cp.start()
                cps.append(cp)
            return cps

        for slot in range(7):
            bsel = slot % 2
            ids_v, out_v = ids_bufs[bsel], out_bufs[bsel]
            nb = slot_params(slot)[1]
            seg = nb * 2 * LANE
            in_copy(slot, False).wait()
            if slot >= 2:
                for cp in out_copies(slot - 2, False):
                    cp.wait()

            @pl.loop(0, nb)
            def _(n, ids_v=ids_v, out_v=out_v, seg=seg):
                for s in range(SUB):
                    off_in = n * 1024 + s * 128
                    off_out = s * seg + n * 256
                    idvs = [ids_v[pl.ds(off_in + u * L, L)]
                            for u in range(8)]
                    for u in range(8):
                        out_v[pl.ds(off_out + u * L, L)] = (
                            _gather(t0, idvs[u]))
                    for u in range(8):
                        out_v[pl.ds(off_out + 128 + u * L, L)] = (
                            _gather(t1, idvs[u]))

            out_copies(slot, True)
            if slot + 2 < 7:
                in_copy(slot + 2, True)

        for slot in (5, 6):
            for cp in out_copies(slot, False):
                cp.wait()

    return run(ids_flat, emb_flat, w_flat, bias)


def kernel(input_ids, embed_weight, lm_head_weight, lm_head_bias):
    ids_flat = (
        input_ids.astype(jnp.int32)
        .reshape(IT, LANE, JT, SUB)
        .transpose(2, 0, 3, 1)
        .reshape(-1)
    )
    emb_flat = embed_weight.T.reshape(-1)
    w_flat = lm_head_weight.reshape(-1)
    out_flat = _sc_lookup(ids_flat, emb_flat, w_flat, lm_head_bias)
    return (
        out_flat.reshape(SEQ, B // 128, 2, 128)
        .transpose(1, 3, 0, 2)
        .reshape(B, SEQ, 2)
    )
